# Initial kernel scaffold; baseline (speedup 1.0000x reference)
import jax
import jax.numpy as jnp
from jax.experimental import pallas as pl


def kernel(x, edge_index, W1, a1_s, a1_d, b1, W2, a2_s, a2_d, b2, W3, a3_s, a3_d, b3, Wp, bp):
    raise NotImplementedError("write your pallas kernel here")


# speedup vs baseline: 1.0353x; 1.0353x over previous
import functools

import jax
import jax.numpy as jnp
from jax.experimental import pallas as pl
from jax.experimental.pallas import tpu as pltpu

N = 10000
NP = 10240
E = 320000
HEADS = 8
HID = 64
BN = 256


def _mm_body(x_ref, w_ref, asd_ref, h_ref, al_ref):
    h = jnp.dot(x_ref[...], w_ref[...], preferred_element_type=jnp.float32)
    h_ref[...] = h
    al_ref[...] = jnp.dot(h, asd_ref[...], preferred_element_type=jnp.float32)


def _mm(xin, W, Asd):
    K = xin.shape[1]
    Kout = W.shape[1]
    grid = NP // BN
    return pl.pallas_call(
        _mm_body,
        grid=(grid,),
        in_specs=[
            pl.BlockSpec((BN, K), lambda i: (i, 0)),
            pl.BlockSpec((K, Kout), lambda i: (0, 0)),
            pl.BlockSpec((Kout, 16), lambda i: (0, 0)),
        ],
        out_specs=[
            pl.BlockSpec((BN, Kout), lambda i: (i, 0)),
            pl.BlockSpec((BN, 16), lambda i: (i, 0)),
        ],
        out_shape=[
            jax.ShapeDtypeStruct((NP, Kout), jnp.float32),
            jax.ShapeDtypeStruct((NP, 16), jnp.float32),
        ],
    )(xin, W, Asd)


def _proj_body(x_ref, w_ref, b_ref, o_ref):
    o_ref[...] = (
        jnp.dot(x_ref[...], w_ref[...], preferred_element_type=jnp.float32)
        + b_ref[...]
    )


def _proj(x3, Wp, bp):
    OUT = Wp.shape[1]
    grid = NP // BN
    return pl.pallas_call(
        _proj_body,
        grid=(grid,),
        in_specs=[
            pl.BlockSpec((BN, x3.shape[1]), lambda i: (i, 0)),
            pl.BlockSpec((x3.shape[1], OUT), lambda i: (0, 0)),
            pl.BlockSpec((1, OUT), lambda i: (0, 0)),
        ],
        out_specs=pl.BlockSpec((BN, OUT), lambda i: (i, 0)),
        out_shape=jax.ShapeDtypeStruct((NP, OUT), jnp.float32),
    )(x3, Wp, bp.reshape(1, OUT))


def _edge_phase(h, alsd, src, dst):
    al_s = alsd[:, :8]
    al_d = alsd[:, 8:]
    e = al_s[src] + al_d[dst]
    e = jnp.where(e > 0, e, 0.2 * e)
    ex = jnp.exp(e)
    denom = jax.ops.segment_sum(ex, dst, num_segments=N)
    alpha = ex / (denom[dst] + 1e-16)
    hh = h[:N].reshape(N, HEADS, HID)
    msg = hh[src] * alpha[:, :, None]
    out = jax.ops.segment_sum(msg, dst, num_segments=N)
    return out


def _make_asd(a_s, a_d):
    Asd = jnp.zeros((HEADS * HID, 16), jnp.float32)
    for hd in range(HEADS):
        Asd = Asd.at[hd * HID:(hd + 1) * HID, hd].set(a_s[hd])
        Asd = Asd.at[hd * HID:(hd + 1) * HID, 8 + hd].set(a_d[hd])
    return Asd


def kernel(x, edge_index, W1, a1_s, a1_d, b1, W2, a2_s, a2_d, b2,
           W3, a3_s, a3_d, b3, Wp, bp):
    src = edge_index[0]
    dst = edge_index[1]
    xp = jnp.pad(x, ((0, NP - N), (0, 0)))

    h1, al1 = _mm(xp, W1, _make_asd(a1_s, a1_d))
    o1 = _edge_phase(h1, al1[:N], src, dst)
    x1 = jax.nn.elu(o1.reshape(N, HEADS * HID) + b1)

    h2, al2 = _mm(jnp.pad(x1, ((0, NP - N), (0, 0))), W2, _make_asd(a2_s, a2_d))
    o2 = _edge_phase(h2, al2[:N], src, dst)
    x2 = jax.nn.elu(o2.reshape(N, HEADS * HID) + b2)

    h3, al3 = _mm(jnp.pad(x2, ((0, NP - N), (0, 0))), W3, _make_asd(a3_s, a3_d))
    o3 = _edge_phase(h3, al3[:N], src, dst)
    x3 = o3.mean(axis=1) + b3

    out = _proj(jnp.pad(x3, ((0, NP - N), (0, 0))), Wp, bp)
    return out[:N]


# speedup vs baseline: 10.1997x; 9.8519x over previous
import functools

import jax
import jax.numpy as jnp
from jax import lax
from jax.experimental import pallas as pl
from jax.experimental.pallas import tpu as pltpu
from jax.experimental.pallas import tpu_sc as plsc

N = 10000
NP = 10240
E = 320000
EPAD = 327680
HEADS = 8
HID = 64
BN = 256
NW = 32
NBLK = 80
NPT = NP // 16
H0S = (0, 2, 4, 6)
GHNS = (2, 2, 2, 2)


def _mm1_body(x_ref, w_ref, asd_ref, h_ref, al_ref):
    h = jnp.dot(x_ref[...], w_ref[...], preferred_element_type=jnp.float32)
    h_ref[...] = h
    al_ref[...] = jnp.dot(h, asd_ref[...], preferred_element_type=jnp.float32)


def _mm1(xp, W, Asd):
    K = xp.shape[1]
    return pl.pallas_call(
        _mm1_body,
        grid=(NP // BN,),
        in_specs=[
            pl.BlockSpec((BN, K), lambda i: (i, 0)),
            pl.BlockSpec((K, 512), lambda i: (0, 0)),
            pl.BlockSpec((512, 16), lambda i: (0, 0)),
        ],
        out_specs=[
            pl.BlockSpec((BN, 512), lambda i: (i, 0)),
            pl.BlockSpec((BN, 16), lambda i: (i, 0)),
        ],
        out_shape=[
            jax.ShapeDtypeStruct((NP, 512), jnp.float32),
            jax.ShapeDtypeStruct((NP, 16), jnp.float32),
        ],
    )(xp, W, Asd)


def _mm2_body(part_ref, b_ref, w_ref, asd_ref, h_ref, al_ref):
    acc = jnp.zeros((BN, 512), jnp.float32)
    for hd in range(HEADS):
        p = part_ref[0, hd] + part_ref[1, hd] + b_ref[hd][None, :]
        xh = jnp.where(p > 0, p, jnp.exp(jnp.minimum(p, 0.0)) - 1.0)
        acc = acc + jnp.dot(xh, w_ref[hd], preferred_element_type=jnp.float32)
    h_ref[...] = acc
    al_ref[...] = jnp.dot(acc, asd_ref[...], preferred_element_type=jnp.float32)


def _mm2(part, b_prev, W, Asd):
    return pl.pallas_call(
        _mm2_body,
        grid=(NP // BN,),
        in_specs=[
            pl.BlockSpec((2, HEADS, BN, HID), lambda i: (0, 0, i, 0)),
            pl.BlockSpec((HEADS, HID), lambda i: (0, 0)),
            pl.BlockSpec((HEADS, HID, 512), lambda i: (0, 0, 0)),
            pl.BlockSpec((512, 16), lambda i: (0, 0)),
        ],
        out_specs=[
            pl.BlockSpec((BN, 512), lambda i: (i, 0)),
            pl.BlockSpec((BN, 16), lambda i: (i, 0)),
        ],
        out_shape=[
            jax.ShapeDtypeStruct((NP, 512), jnp.float32),
            jax.ShapeDtypeStruct((NP, 16), jnp.float32),
        ],
    )(part, b_prev, W, Asd)


def _inv_body(p_ref, o_ref):
    o_ref[...] = 1.0 / (p_ref[0] + p_ref[1] + 1e-16)


def _inv(part):
    p = part.reshape(2, NP // 16, 256)
    out = pl.pallas_call(
        _inv_body,
        grid=(5,),
        in_specs=[pl.BlockSpec((2, NP // 80, 256), lambda i: (0, i, 0))],
        out_specs=pl.BlockSpec((NP // 80, 256), lambda i: (i, 0)),
        out_shape=jax.ShapeDtypeStruct((NP // 16, 256), jnp.float32),
    )(p)
    return out.reshape(NP, 16)


def _proj_body(part_ref, b3_ref, w_ref, bp_ref, o_ref):
    x3 = jnp.zeros((BN, HID), jnp.float32)
    for hd in range(HEADS):
        x3 = x3 + part_ref[0, hd] + part_ref[1, hd]
    x3 = x3 * (1.0 / HEADS) + b3_ref[...]
    o_ref[...] = (
        jnp.dot(x3, w_ref[...], preferred_element_type=jnp.float32)
        + bp_ref[...]
    )


def _proj(part, b3, Wp, bp):
    OUT = Wp.shape[1]
    return pl.pallas_call(
        _proj_body,
        grid=(NP // BN,),
        in_specs=[
            pl.BlockSpec((2, HEADS, BN, HID), lambda i: (0, 0, i, 0)),
            pl.BlockSpec((1, HID), lambda i: (0, 0)),
            pl.BlockSpec((HID, OUT), lambda i: (0, 0)),
            pl.BlockSpec((1, OUT), lambda i: (0, 0)),
        ],
        out_specs=pl.BlockSpec((BN, OUT), lambda i: (i, 0)),
        out_shape=jax.ShapeDtypeStruct((NP, OUT), jnp.float32),
    )(part, b3.reshape(1, HID), Wp, bp.reshape(1, OUT))


_MESH = plsc.VectorSubcoreMesh(
    core_axis_name="c", subcore_axis_name="s", num_cores=2, num_subcores=16)


def _attn_body(asd_hbm, srcv_hbm, dstv_hbm, z16_hbm,
               ex_hbm, part_hbm,
               denom_acc, srcb, dstb, srows, drows, exb, sem):
    cid = lax.axis_index("c")
    sid = lax.axis_index("s")
    wid = sid * 2 + cid

    iota = lax.iota(jnp.int32, 16)
    row_half = iota >> 3
    col8 = iota & 7
    zeros16 = jnp.zeros((16,), jnp.float32)

    pltpu.sync_copy(srcv_hbm.at[wid], srcb)
    pltpu.sync_copy(dstv_hbm.at[wid], dstb)

    for v in range(64):
        plsc.store_scatter(exb, [row_half + 2 * v, col8 + 8], zeros16)

    pltpu.sync_copy(z16_hbm, denom_acc.at[pl.ds(sid * NPT, NPT)])
    plsc.subcore_barrier()

    def block(b, _):
        pltpu.async_copy(asd_hbm.at[srcb.at[b]], srows, sem).wait()
        pltpu.async_copy(asd_hbm.at[dstb.at[b]], drows, sem).wait()
        for v in range(64):
            row = row_half + 2 * v
            es = plsc.load_gather(srows, [row, col8])
            ed = plsc.load_gather(drows, [row, col8 + 8])
            e = es + ed
            e = jnp.where(e > 0, e, 0.2 * e)
            plsc.store_scatter(exb, [row, col8], jnp.exp(e))
        pltpu.sync_copy(exb, ex_hbm.at[wid, b])
        pltpu.sync_copy(exb, denom_acc.at[dstb.at[b]], add=True)
        return _

    lax.fori_loop(0, NBLK, block, None)

    plsc.subcore_barrier()
    pltpu.sync_copy(denom_acc.at[pl.ds(sid * NPT, NPT)],
                    part_hbm.at[cid, pl.ds(sid * NPT, NPT)])


@functools.partial(jax.jit, static_argnums=())
def _attn(asd, srcv, dstv, z16):
    return pl.kernel(
        _attn_body,
        out_type=[
            jax.ShapeDtypeStruct((NW, NBLK, 128, 16), jnp.float32),
            jax.ShapeDtypeStruct((2, NP, 16), jnp.float32),
        ],
        mesh=_MESH,
        compiler_params=pltpu.CompilerParams(needs_layout_passes=False, use_tc_tiling_on_sc=False),
        scratch_types=[
            pltpu.VMEM_SHARED((NP, 16), jnp.float32),
            pltpu.VMEM((NBLK, 128), jnp.int32),
            pltpu.VMEM((NBLK, 128), jnp.int32),
            pltpu.VMEM((128, 16), jnp.float32),
            pltpu.VMEM((128, 16), jnp.float32),
            pltpu.VMEM((128, 16), jnp.float32),
            pltpu.SemaphoreType.DMA,
        ],
    )(asd, srcv, dstv, z16)


def _agg_body(hview_hbm, ex_hbm, inv_hbm, srcv_hbm, dstv_hbm, zbig_hbm,
              part_hbm,
              acc, srcb, dstb, idxh, idxo, exb, invb, hrows, sem):
    cid = lax.axis_index("c")
    sid = lax.axis_index("s")
    wid = sid * 2 + cid

    iota = lax.iota(jnp.int32, 16)
    row_half = iota >> 3
    col8 = iota & 7

    for g in range(len(H0S)):
        h0 = H0S[g]
        ghn = GHNS[g]

        for gp in range(ghn):
            pltpu.sync_copy(zbig_hbm,
                            acc.at[pl.ds(gp * NP + sid * NPT, NPT)])
        plsc.subcore_barrier()

        def block(b, _, h0=h0, ghn=ghn):
            pltpu.sync_copy(srcv_hbm.at[wid, b], srcb.at[0])
            pltpu.sync_copy(dstv_hbm.at[wid, b], dstb.at[0])
            pltpu.sync_copy(ex_hbm.at[wid, b], exb)
            pltpu.async_copy(inv_hbm.at[dstb.at[0]], invb, sem).wait()
            for v in range(64):
                row = row_half + 2 * v
                xv = plsc.load_gather(exb, [row, col8])
                iv = plsc.load_gather(invb, [row, col8])
                plsc.store_scatter(exb, [row, col8], xv * iv)
            for gp in range(ghn):
                for c in range(8):
                    s16 = srcb[0, pl.ds(c * 16, 16)]
                    d16 = dstb[0, pl.ds(c * 16, 16)]
                    idxh[gp, pl.ds(c * 16, 16)] = (s16 << 3) + (h0 + gp)
                    idxo[gp, pl.ds(c * 16, 16)] = d16 + gp * NP
            cps = [pltpu.async_copy(hview_hbm.at[idxh.at[gp]],
                                    hrows.at[gp], sem)
                   for gp in range(ghn)]
            for cp in cps:
                cp.wait()

            def scale(j, _):
                for gp in range(ghn):
                    w = plsc.load_gather(
                        exb, [jnp.full((16,), j, jnp.int32),
                              jnp.full((16,), h0 + gp, jnp.int32)])
                    for c in range(4):
                        sl = hrows[gp, j, pl.ds(c * 16, 16)]
                        hrows[gp, j, pl.ds(c * 16, 16)] = sl * w
                return _

            lax.fori_loop(0, 128, scale, None)

            for gp in range(ghn):
                pltpu.sync_copy(hrows.at[gp], acc.at[idxo.at[gp]], add=True)
            return _

        lax.fori_loop(0, NBLK, block, None)

        plsc.subcore_barrier()
        for gp in range(ghn):
            pltpu.sync_copy(
                acc.at[pl.ds(gp * NP + sid * NPT, NPT)],
                part_hbm.at[cid, h0 + gp, pl.ds(sid * NPT, NPT)])
        plsc.subcore_barrier()


def _agg(hview, ex, inv16, srcv, dstv, zbig):
    return pl.kernel(
        _agg_body,
        out_type=jax.ShapeDtypeStruct((2, HEADS, NP, HID), jnp.float32),
        mesh=_MESH,
        compiler_params=pltpu.CompilerParams(needs_layout_passes=False, use_tc_tiling_on_sc=False),
        scratch_types=[
            pltpu.VMEM_SHARED((2 * NP, HID), jnp.float32),
            pltpu.VMEM((1, 128), jnp.int32),
            pltpu.VMEM((1, 128), jnp.int32),
            pltpu.VMEM((2, 128), jnp.int32),
            pltpu.VMEM((2, 128), jnp.int32),
            pltpu.VMEM((128, 16), jnp.float32),
            pltpu.VMEM((128, 16), jnp.float32),
            pltpu.VMEM((2, 128, HID), jnp.float32),
            pltpu.SemaphoreType.DMA,
        ],
    )(hview, ex, inv16, srcv, dstv, zbig)


def _make_asd(a_s, a_d):
    eye = jnp.eye(HEADS, dtype=jnp.float32)
    As = (eye[:, None, :] * a_s[:, :, None]).reshape(HEADS * HID, HEADS)
    Ad = (eye[:, None, :] * a_d[:, :, None]).reshape(HEADS * HID, HEADS)
    return jnp.concatenate([As, Ad], axis=1)


def _layer_edges(h, asd, srcv, dstv, z16, zbig):
    ex, parts = _attn(asd, srcv, dstv, z16)
    inv16 = _inv(parts)
    return _agg(h.reshape(NP * HEADS, HID), ex, inv16, srcv, dstv, zbig)


def kernel(x, edge_index, W1, a1_s, a1_d, b1, W2, a2_s, a2_d, b2,
           W3, a3_s, a3_d, b3, Wp, bp):
    src = jnp.concatenate(
        [edge_index[0], jnp.full((EPAD - E,), N, jnp.int32)]).reshape(
            NW, NBLK, 128)
    dst = jnp.concatenate(
        [edge_index[1], jnp.full((EPAD - E,), N, jnp.int32)]).reshape(
            NW, NBLK, 128)
    z16 = jnp.zeros((NPT, 16), jnp.float32)
    zbig = jnp.zeros((NPT, HID), jnp.float32)
    xp = jnp.pad(x, ((0, NP - N), (0, 0)))

    h1, al1 = _mm1(xp, W1, _make_asd(a1_s, a1_d))
    p1 = _layer_edges(h1, al1, src, dst, z16, zbig)

    h2, al2 = _mm2(p1, b1.reshape(HEADS, HID), W2.reshape(HEADS, HID, 512),
                   _make_asd(a2_s, a2_d))
    p2 = _layer_edges(h2, al2, src, dst, z16, zbig)

    h3, al3 = _mm2(p2, b2.reshape(HEADS, HID), W3.reshape(HEADS, HID, 512),
                   _make_asd(a3_s, a3_d))
    p3 = _layer_edges(h3, al3, src, dst, z16, zbig)

    out = _proj(p3, b3, Wp, bp)
    return out[:N]


# speedup vs baseline: 16.8642x; 1.6534x over previous
import functools

import jax
import jax.numpy as jnp
from jax import lax
from jax.experimental import pallas as pl
from jax.experimental.pallas import tpu as pltpu
from jax.experimental.pallas import tpu_sc as plsc

N = 10000
NP = 10240
E = 320000
EPAD = 327680
HEADS = 8
HID = 64
BN = 256
NW = 32
NBLK = 80
NPT = NP // 16
H0S = (0, 2, 4, 6)
GHNS = (2, 2, 2, 2)


def _mm1_body(x_ref, w_ref, asd_ref, h_ref, al_ref):
    h = jnp.dot(x_ref[...], w_ref[...], preferred_element_type=jnp.float32)
    h_ref[...] = h
    al_ref[...] = jnp.dot(h, asd_ref[...], preferred_element_type=jnp.float32)


def _mm1(xp, W, Asd):
    K = xp.shape[1]
    return pl.pallas_call(
        _mm1_body,
        grid=(NP // BN,),
        in_specs=[
            pl.BlockSpec((BN, K), lambda i: (i, 0)),
            pl.BlockSpec((K, 512), lambda i: (0, 0)),
            pl.BlockSpec((512, 16), lambda i: (0, 0)),
        ],
        out_specs=[
            pl.BlockSpec((BN, 512), lambda i: (i, 0)),
            pl.BlockSpec((BN, 16), lambda i: (i, 0)),
        ],
        out_shape=[
            jax.ShapeDtypeStruct((NP, 512), jnp.float32),
            jax.ShapeDtypeStruct((NP, 16), jnp.float32),
        ],
    )(xp, W, Asd)


def _mm2_body(part_ref, b_ref, w_ref, asd_ref, h_ref, al_ref):
    acc = jnp.zeros((BN, 512), jnp.float32)
    for hd in range(HEADS):
        p = part_ref[0, hd] + part_ref[1, hd] + b_ref[hd][None, :]
        xh = jnp.where(p > 0, p, jnp.exp(jnp.minimum(p, 0.0)) - 1.0)
        acc = acc + jnp.dot(xh, w_ref[hd], preferred_element_type=jnp.float32)
    h_ref[...] = acc
    al_ref[...] = jnp.dot(acc, asd_ref[...], preferred_element_type=jnp.float32)


def _mm2(part, b_prev, W, Asd):
    return pl.pallas_call(
        _mm2_body,
        grid=(NP // BN,),
        in_specs=[
            pl.BlockSpec((2, HEADS, BN, HID), lambda i: (0, 0, i, 0)),
            pl.BlockSpec((HEADS, HID), lambda i: (0, 0)),
            pl.BlockSpec((HEADS, HID, 512), lambda i: (0, 0, 0)),
            pl.BlockSpec((512, 16), lambda i: (0, 0)),
        ],
        out_specs=[
            pl.BlockSpec((BN, 512), lambda i: (i, 0)),
            pl.BlockSpec((BN, 16), lambda i: (i, 0)),
        ],
        out_shape=[
            jax.ShapeDtypeStruct((NP, 512), jnp.float32),
            jax.ShapeDtypeStruct((NP, 16), jnp.float32),
        ],
    )(part, b_prev, W, Asd)


def _inv_body(p_ref, o_ref):
    o_ref[...] = 1.0 / (p_ref[0] + p_ref[1] + 1e-16)


def _inv(part):
    p = part.reshape(2, NP // 16, 256)
    out = pl.pallas_call(
        _inv_body,
        grid=(5,),
        in_specs=[pl.BlockSpec((2, NP // 80, 256), lambda i: (0, i, 0))],
        out_specs=pl.BlockSpec((NP // 80, 256), lambda i: (i, 0)),
        out_shape=jax.ShapeDtypeStruct((NP // 16, 256), jnp.float32),
    )(p)
    return out.reshape(NP, 16)


def _proj_body(part_ref, b3_ref, w_ref, bp_ref, o_ref):
    x3 = jnp.zeros((BN, HID), jnp.float32)
    for hd in range(HEADS):
        x3 = x3 + part_ref[0, hd] + part_ref[1, hd]
    x3 = x3 * (1.0 / HEADS) + b3_ref[...]
    o_ref[...] = (
        jnp.dot(x3, w_ref[...], preferred_element_type=jnp.float32)
        + bp_ref[...]
    )


def _proj(part, b3, Wp, bp):
    OUT = Wp.shape[1]
    return pl.pallas_call(
        _proj_body,
        grid=(NP // BN,),
        in_specs=[
            pl.BlockSpec((2, HEADS, BN, HID), lambda i: (0, 0, i, 0)),
            pl.BlockSpec((1, HID), lambda i: (0, 0)),
            pl.BlockSpec((HID, OUT), lambda i: (0, 0)),
            pl.BlockSpec((1, OUT), lambda i: (0, 0)),
        ],
        out_specs=pl.BlockSpec((BN, OUT), lambda i: (i, 0)),
        out_shape=jax.ShapeDtypeStruct((NP, OUT), jnp.float32),
    )(part, b3.reshape(1, HID), Wp, bp.reshape(1, OUT))


_MESH = plsc.VectorSubcoreMesh(
    core_axis_name="c", subcore_axis_name="s", num_cores=2, num_subcores=16)


def _attn_body(asd_hbm, srcv_hbm, dstv_hbm, z16_hbm,
               ex_hbm, part_hbm,
               denom_acc, srcb, dstb, srows, drows, exb, sem):
    cid = lax.axis_index("c")
    sid = lax.axis_index("s")
    wid = sid * 2 + cid

    iota = lax.iota(jnp.int32, 16)
    row_half = iota >> 3
    col8 = iota & 7
    zeros16 = jnp.zeros((16,), jnp.float32)

    pltpu.sync_copy(srcv_hbm.at[wid], srcb)
    pltpu.sync_copy(dstv_hbm.at[wid], dstb)

    for v in range(64):
        plsc.store_scatter(exb, [row_half + 2 * v, col8 + 8], zeros16)

    pltpu.sync_copy(z16_hbm, denom_acc.at[pl.ds(sid * NPT, NPT)])
    plsc.subcore_barrier()

    def block(b, _):
        pltpu.async_copy(asd_hbm.at[srcb.at[b]], srows, sem).wait()
        pltpu.async_copy(asd_hbm.at[dstb.at[b]], drows, sem).wait()
        for v in range(64):
            row = row_half + 2 * v
            es = plsc.load_gather(srows, [row, col8])
            ed = plsc.load_gather(drows, [row, col8 + 8])
            e = es + ed
            e = jnp.where(e > 0, e, 0.2 * e)
            plsc.store_scatter(exb, [row, col8], jnp.exp(e))
        pltpu.sync_copy(exb, ex_hbm.at[wid, b])
        pltpu.sync_copy(exb, denom_acc.at[dstb.at[b]], add=True)
        return _

    lax.fori_loop(0, NBLK, block, None)

    plsc.subcore_barrier()
    pltpu.sync_copy(denom_acc.at[pl.ds(sid * NPT, NPT)],
                    part_hbm.at[cid, pl.ds(sid * NPT, NPT)])


@functools.partial(jax.jit, static_argnums=())
def _attn(asd, srcv, dstv, z16):
    return pl.kernel(
        _attn_body,
        out_type=[
            jax.ShapeDtypeStruct((NW, NBLK, 128, 16), jnp.float32),
            jax.ShapeDtypeStruct((2, NP, 16), jnp.float32),
        ],
        mesh=_MESH,
        compiler_params=pltpu.CompilerParams(needs_layout_passes=False, use_tc_tiling_on_sc=False),
        scratch_types=[
            pltpu.VMEM_SHARED((NP, 16), jnp.float32),
            pltpu.VMEM((NBLK, 128), jnp.int32),
            pltpu.VMEM((NBLK, 128), jnp.int32),
            pltpu.VMEM((128, 16), jnp.float32),
            pltpu.VMEM((128, 16), jnp.float32),
            pltpu.VMEM((128, 16), jnp.float32),
            pltpu.SemaphoreType.DMA,
        ],
    )(asd, srcv, dstv, z16)


def _half_agg(hview_hbm, ex_hbm, inv_hbm, srcv_hbm, dstv_hbm, part_hbm, acc,
              CUR, NXT, wid, b, h0, guard_sw, guard_cons, row_half, col8):
    (c_src, c_dst, c_ixh, c_ixo, c_ex, c_inv, c_hr, c_w1, c_w2, c_s) = CUR
    (n_src, n_dst, n_ixh, n_ixo, n_ex, n_inv, n_hr, n_w1, n_w2, n_s) = NXT

    pltpu.make_async_copy(srcv_hbm.at[wid, 0], c_src.at[0], c_w1).wait()
    pltpu.make_async_copy(dstv_hbm.at[wid, 0], c_dst.at[0], c_w1).wait()
    pltpu.make_async_copy(ex_hbm.at[wid, 0], c_ex, c_w1).wait()

    def _wait_scatter():
        for gp in range(2):
            pltpu.make_async_copy(c_hr.at[gp], acc.at[c_ixo.at[gp]], c_s).wait()
    if guard_sw is None:
        _wait_scatter()
    else:
        pl.when(guard_sw)(_wait_scatter)

    for gp in range(2):
        for c in range(8):
            s16 = c_src[0, pl.ds(c * 16, 16)]
            d16 = c_dst[0, pl.ds(c * 16, 16)]
            c_ixh[gp, pl.ds(c * 16, 16)] = (s16 << 3) + (h0 + gp)
            c_ixo[gp, pl.ds(c * 16, 16)] = d16 + gp * NP

    pltpu.async_copy(inv_hbm.at[c_dst.at[0]], c_inv, c_w2)
    for gp in range(2):
        pltpu.async_copy(hview_hbm.at[c_ixh.at[gp]], c_hr.at[gp], c_w2)

    def _consume():
        pltpu.make_async_copy(inv_hbm.at[n_dst.at[0]], n_inv, n_w2).wait()
        for gp in range(2):
            pltpu.make_async_copy(hview_hbm.at[n_ixh.at[gp]], n_hr.at[gp],
                                  n_w2).wait()
        for v in range(64):
            row = row_half + 2 * v
            xv = plsc.load_gather(n_ex, [row, col8])
            iv = plsc.load_gather(n_inv, [row, col8])
            plsc.store_scatter(n_ex, [row, col8], xv * iv)

        @plsc.parallel_loop(0, 128, 1, unroll=2)
        def _scale(j):
            for gp in range(2):
                w = plsc.load_gather(
                    n_ex, [jnp.full((16,), j, jnp.int32),
                           jnp.full((16,), h0 + gp, jnp.int32)])
                for c in range(4):
                    sl = n_hr[gp, j, pl.ds(c * 16, 16)]
                    n_hr[gp, j, pl.ds(c * 16, 16)] = sl * w

        for gp in range(2):
            pltpu.async_copy(n_hr.at[gp], acc.at[n_ixo.at[gp]], n_s, add=True)

    if guard_cons is None:
        _consume()
    else:
        pl.when(guard_cons)(_consume)

    bn = jnp.minimum(b + 1, NBLK - 1)
    pltpu.async_copy(srcv_hbm.at[wid, bn], n_src.at[0], n_w1)
    pltpu.async_copy(dstv_hbm.at[wid, bn], n_dst.at[0], n_w1)
    pltpu.async_copy(ex_hbm.at[wid, bn], n_ex, n_w1)


def _agg_body(hview_hbm, ex_hbm, inv_hbm, srcv_hbm, dstv_hbm, zbig_hbm,
              part_hbm, acc,
              srcA, dstA, ixhA, ixoA, exA, invA, hrA, w1A, w2A, sA,
              srcB, dstB, ixhB, ixoB, exB, invB, hrB, w1B, w2B, sB):
    cid = lax.axis_index("c")
    sid = lax.axis_index("s")
    wid = sid * 2 + cid

    iota = lax.iota(jnp.int32, 16)
    row_half = iota >> 3
    col8 = iota & 7

    A = (srcA, dstA, ixhA, ixoA, exA, invA, hrA, w1A, w2A, sA)
    B = (srcB, dstB, ixhB, ixoB, exB, invB, hrB, w1B, w2B, sB)

    for g in range(len(H0S)):
        h0 = H0S[g]

        for gp in range(2):
            pltpu.sync_copy(zbig_hbm,
                            acc.at[pl.ds(gp * NP + sid * NPT, NPT)])
        plsc.subcore_barrier()

        pltpu.async_copy(srcv_hbm.at[wid, 0], srcA.at[0], w1A)
        pltpu.async_copy(dstv_hbm.at[wid, 0], dstA.at[0], w1A)
        pltpu.async_copy(ex_hbm.at[wid, 0], exA, w1A)

        def pair(k, _, h0=h0):
            _half_agg(hview_hbm, ex_hbm, inv_hbm, srcv_hbm, dstv_hbm,
                      part_hbm, acc, A, B, wid, 2 * k, h0,
                      k > 0, k > 0, row_half, col8)
            _half_agg(hview_hbm, ex_hbm, inv_hbm, srcv_hbm, dstv_hbm,
                      part_hbm, acc, B, A, wid, 2 * k + 1, h0,
                      k > 0, None, row_half, col8)
            return _

        lax.fori_loop(0, NBLK // 2, pair, None)

        pltpu.make_async_copy(inv_hbm.at[dstB.at[0]], invB, w2B).wait()
        for gp in range(2):
            pltpu.make_async_copy(hview_hbm.at[ixhB.at[gp]], hrB.at[gp],
                                  w2B).wait()
        for v in range(64):
            row = row_half + 2 * v
            xv = plsc.load_gather(exB, [row, col8])
            iv = plsc.load_gather(invB, [row, col8])
            plsc.store_scatter(exB, [row, col8], xv * iv)

        @plsc.parallel_loop(0, 128, 1, unroll=2)
        def _scale_tail(j, h0=h0):
            for gp in range(2):
                w = plsc.load_gather(
                    exB, [jnp.full((16,), j, jnp.int32),
                          jnp.full((16,), h0 + gp, jnp.int32)])
                for c in range(4):
                    sl = hrB[gp, j, pl.ds(c * 16, 16)]
                    hrB[gp, j, pl.ds(c * 16, 16)] = sl * w

        for gp in range(2):
            pltpu.async_copy(hrB.at[gp], acc.at[ixoB.at[gp]], sB, add=True)

        pltpu.make_async_copy(srcv_hbm.at[wid, 0], srcA.at[0], w1A).wait()
        pltpu.make_async_copy(dstv_hbm.at[wid, 0], dstA.at[0], w1A).wait()
        pltpu.make_async_copy(ex_hbm.at[wid, 0], exA, w1A).wait()
        for gp in range(2):
            pltpu.make_async_copy(hrA.at[gp], acc.at[ixoA.at[gp]], sA).wait()
        for gp in range(2):
            pltpu.make_async_copy(hrB.at[gp], acc.at[ixoB.at[gp]], sB).wait()

        plsc.subcore_barrier()
        for gp in range(2):
            pltpu.sync_copy(
                acc.at[pl.ds(gp * NP + sid * NPT, NPT)],
                part_hbm.at[cid, h0 + gp, pl.ds(sid * NPT, NPT)])
        plsc.subcore_barrier()


def _agg(hview, ex, inv16, srcv, dstv, zbig):
    buf = lambda: [
        pltpu.VMEM((1, 128), jnp.int32),
        pltpu.VMEM((1, 128), jnp.int32),
        pltpu.VMEM((2, 128), jnp.int32),
        pltpu.VMEM((2, 128), jnp.int32),
        pltpu.VMEM((128, 16), jnp.float32),
        pltpu.VMEM((128, 16), jnp.float32),
        pltpu.VMEM((2, 128, HID), jnp.float32),
        pltpu.SemaphoreType.DMA,
        pltpu.SemaphoreType.DMA,
        pltpu.SemaphoreType.DMA,
    ]
    return pl.kernel(
        _agg_body,
        out_type=jax.ShapeDtypeStruct((2, HEADS, NP, HID), jnp.float32),
        mesh=_MESH,
        compiler_params=pltpu.CompilerParams(needs_layout_passes=False, use_tc_tiling_on_sc=False),
        scratch_types=[pltpu.VMEM_SHARED((2 * NP, HID), jnp.float32)]
        + buf() + buf(),
    )(hview, ex, inv16, srcv, dstv, zbig)


def _make_asd(a_s, a_d):
    eye = jnp.eye(HEADS, dtype=jnp.float32)
    As = (eye[:, None, :] * a_s[:, :, None]).reshape(HEADS * HID, HEADS)
    Ad = (eye[:, None, :] * a_d[:, :, None]).reshape(HEADS * HID, HEADS)
    return jnp.concatenate([As, Ad], axis=1)


def _layer_edges(h, asd, srcv, dstv, z16, zbig):
    ex, parts = _attn(asd, srcv, dstv, z16)
    inv16 = _inv(parts)
    return _agg(h.reshape(NP * HEADS, HID), ex, inv16, srcv, dstv, zbig)


def kernel(x, edge_index, W1, a1_s, a1_d, b1, W2, a2_s, a2_d, b2,
           W3, a3_s, a3_d, b3, Wp, bp):
    src = jnp.concatenate(
        [edge_index[0], jnp.full((EPAD - E,), N, jnp.int32)]).reshape(
            NW, NBLK, 128)
    dst = jnp.concatenate(
        [edge_index[1], jnp.full((EPAD - E,), N, jnp.int32)]).reshape(
            NW, NBLK, 128)
    z16 = jnp.zeros((NPT, 16), jnp.float32)
    zbig = jnp.zeros((NPT, HID), jnp.float32)
    xp = jnp.pad(x, ((0, NP - N), (0, 0)))

    h1, al1 = _mm1(xp, W1, _make_asd(a1_s, a1_d))
    p1 = _layer_edges(h1, al1, src, dst, z16, zbig)

    h2, al2 = _mm2(p1, b1.reshape(HEADS, HID), W2.reshape(HEADS, HID, 512),
                   _make_asd(a2_s, a2_d))
    p2 = _layer_edges(h2, al2, src, dst, z16, zbig)

    h3, al3 = _mm2(p2, b2.reshape(HEADS, HID), W3.reshape(HEADS, HID, 512),
                   _make_asd(a3_s, a3_d))
    p3 = _layer_edges(h3, al3, src, dst, z16, zbig)

    out = _proj(p3, b3, Wp, bp)
    return out[:N]


# speedup vs baseline: 37.3690x; 2.2159x over previous
import functools

import jax
import jax.numpy as jnp
from jax import lax
from jax.experimental import pallas as pl
from jax.experimental.pallas import tpu as pltpu
from jax.experimental.pallas import tpu_sc as plsc

N = 10000
NP = 10240
E = 320000
EPAD = 327680
HEADS = 8
HID = 64
BN = 256
NW = 32
NBLK = 80
NPT = NP // 16
H0S = (0, 2, 4, 6)
GHNS = (2, 2, 2, 2)


def _mm1_body(x_ref, w_ref, asd_ref, h_ref, al_ref):
    h = jnp.dot(x_ref[...], w_ref[...], preferred_element_type=jnp.float32)
    h_ref[...] = h
    al_ref[...] = jnp.dot(h, asd_ref[...], preferred_element_type=jnp.float32)


def _mm1(xp, W, Asd):
    K = xp.shape[1]
    return pl.pallas_call(
        _mm1_body,
        grid=(NP // BN,),
        in_specs=[
            pl.BlockSpec((BN, K), lambda i: (i, 0)),
            pl.BlockSpec((K, 512), lambda i: (0, 0)),
            pl.BlockSpec((512, 16), lambda i: (0, 0)),
        ],
        out_specs=[
            pl.BlockSpec((BN, 512), lambda i: (i, 0)),
            pl.BlockSpec((BN, 16), lambda i: (i, 0)),
        ],
        out_shape=[
            jax.ShapeDtypeStruct((NP, 512), jnp.float32),
            jax.ShapeDtypeStruct((NP, 16), jnp.float32),
        ],
    )(xp, W, Asd)


def _mm2_body(part_ref, b_ref, w_ref, asd_ref, h_ref, al_ref):
    acc = jnp.zeros((BN, 512), jnp.float32)
    for hd in range(HEADS):
        p = part_ref[0, hd] + part_ref[1, hd] + b_ref[hd][None, :]
        xh = jnp.where(p > 0, p, jnp.exp(jnp.minimum(p, 0.0)) - 1.0)
        acc = acc + jnp.dot(xh, w_ref[hd], preferred_element_type=jnp.float32)
    h_ref[...] = acc
    al_ref[...] = jnp.dot(acc, asd_ref[...], preferred_element_type=jnp.float32)


def _mm2(part, b_prev, W, Asd):
    return pl.pallas_call(
        _mm2_body,
        grid=(NP // BN,),
        in_specs=[
            pl.BlockSpec((2, HEADS, BN, HID), lambda i: (0, 0, i, 0)),
            pl.BlockSpec((HEADS, HID), lambda i: (0, 0)),
            pl.BlockSpec((HEADS, HID, 512), lambda i: (0, 0, 0)),
            pl.BlockSpec((512, 16), lambda i: (0, 0)),
        ],
        out_specs=[
            pl.BlockSpec((BN, 512), lambda i: (i, 0)),
            pl.BlockSpec((BN, 16), lambda i: (i, 0)),
        ],
        out_shape=[
            jax.ShapeDtypeStruct((NP, 512), jnp.float32),
            jax.ShapeDtypeStruct((NP, 16), jnp.float32),
        ],
    )(part, b_prev, W, Asd)


def _inv_body(p_ref, o_ref):
    o_ref[...] = 1.0 / (p_ref[0] + p_ref[1] + 1e-16)


def _inv(part):
    p = part.reshape(2, NP // 16, 256)
    out = pl.pallas_call(
        _inv_body,
        grid=(5,),
        in_specs=[pl.BlockSpec((2, NP // 80, 256), lambda i: (0, i, 0))],
        out_specs=pl.BlockSpec((NP // 80, 256), lambda i: (i, 0)),
        out_shape=jax.ShapeDtypeStruct((NP // 16, 256), jnp.float32),
    )(p)
    return out.reshape(NP, 16)


def _proj_body(part_ref, b3_ref, w_ref, bp_ref, o_ref):
    x3 = jnp.zeros((BN, HID), jnp.float32)
    for hd in range(HEADS):
        x3 = x3 + part_ref[0, hd] + part_ref[1, hd]
    x3 = x3 * (1.0 / HEADS) + b3_ref[...]
    o_ref[...] = (
        jnp.dot(x3, w_ref[...], preferred_element_type=jnp.float32)
        + bp_ref[...]
    )


def _proj(part, b3, Wp, bp):
    OUT = Wp.shape[1]
    return pl.pallas_call(
        _proj_body,
        grid=(NP // BN,),
        in_specs=[
            pl.BlockSpec((2, HEADS, BN, HID), lambda i: (0, 0, i, 0)),
            pl.BlockSpec((1, HID), lambda i: (0, 0)),
            pl.BlockSpec((HID, OUT), lambda i: (0, 0)),
            pl.BlockSpec((1, OUT), lambda i: (0, 0)),
        ],
        out_specs=pl.BlockSpec((BN, OUT), lambda i: (i, 0)),
        out_shape=jax.ShapeDtypeStruct((NP, OUT), jnp.float32),
    )(part, b3.reshape(1, HID), Wp, bp.reshape(1, OUT))


_MESH = plsc.VectorSubcoreMesh(
    core_axis_name="c", subcore_axis_name="s", num_cores=2, num_subcores=16)


def _attn_body(asd_hbm, srcv_hbm, dstv_hbm, z16_hbm,
               ex_hbm, part_hbm,
               denom_acc, srcb, dstb, srows, drows, exb, sem):
    cid = lax.axis_index("c")
    sid = lax.axis_index("s")
    wid = sid * 2 + cid

    iota = lax.iota(jnp.int32, 16)
    row_half = iota >> 3
    col8 = iota & 7
    zeros16 = jnp.zeros((16,), jnp.float32)

    pltpu.sync_copy(srcv_hbm.at[wid], srcb)
    pltpu.sync_copy(dstv_hbm.at[wid], dstb)

    for v in range(64):
        plsc.store_scatter(exb, [row_half + 2 * v, col8 + 8], zeros16)

    pltpu.sync_copy(z16_hbm, denom_acc.at[pl.ds(sid * NPT, NPT)])
    plsc.subcore_barrier()

    def block(b, _):
        pltpu.async_copy(asd_hbm.at[srcb.at[b]], srows, sem).wait()
        pltpu.async_copy(asd_hbm.at[dstb.at[b]], drows, sem).wait()
        for v in range(64):
            row = row_half + 2 * v
            es = plsc.load_gather(srows, [row, col8])
            ed = plsc.load_gather(drows, [row, col8 + 8])
            e = es + ed
            e = jnp.where(e > 0, e, 0.2 * e)
            plsc.store_scatter(exb, [row, col8], jnp.exp(e))
        pltpu.sync_copy(exb, ex_hbm.at[wid, b])
        pltpu.sync_copy(exb, denom_acc.at[dstb.at[b]], add=True)
        return _

    lax.fori_loop(0, NBLK, block, None)

    plsc.subcore_barrier()
    pltpu.sync_copy(denom_acc.at[pl.ds(sid * NPT, NPT)],
                    part_hbm.at[cid, pl.ds(sid * NPT, NPT)])


@functools.partial(jax.jit, static_argnums=())
def _attn(asd, srcv, dstv, z16):
    return pl.kernel(
        _attn_body,
        out_type=[
            jax.ShapeDtypeStruct((NW, NBLK, 128, 16), jnp.float32),
            jax.ShapeDtypeStruct((2, NP, 16), jnp.float32),
        ],
        mesh=_MESH,
        compiler_params=pltpu.CompilerParams(needs_layout_passes=False, use_tc_tiling_on_sc=False),
        scratch_types=[
            pltpu.VMEM_SHARED((NP, 16), jnp.float32),
            pltpu.VMEM((NBLK, 128), jnp.int32),
            pltpu.VMEM((NBLK, 128), jnp.int32),
            pltpu.VMEM((128, 16), jnp.float32),
            pltpu.VMEM((128, 16), jnp.float32),
            pltpu.VMEM((128, 16), jnp.float32),
            pltpu.SemaphoreType.DMA,
        ],
    )(asd, srcv, dstv, z16)


def _half_agg(hview_hbm, ex_hbm, inv_hbm, srcv_hbm, dstv_hbm, part_hbm, acc,
              CUR, NXT, wid, b, h0, guard_sw, guard_cons, row_half, col8):
    (c_src, c_dst, c_ixh, c_ixo, c_ex, c_inv, c_hr, c_w1, c_w2, c_s) = CUR
    (n_src, n_dst, n_ixh, n_ixo, n_ex, n_inv, n_hr, n_w1, n_w2, n_s) = NXT

    pltpu.make_async_copy(srcv_hbm.at[wid, 0], c_src.at[0], c_w1).wait()
    pltpu.make_async_copy(dstv_hbm.at[wid, 0], c_dst.at[0], c_w1).wait()
    pltpu.make_async_copy(ex_hbm.at[wid, 0], c_ex, c_w1).wait()

    def _wait_scatter():
        for gp in range(2):
            pltpu.make_async_copy(c_hr.at[gp], acc.at[c_ixo.at[gp]], c_s).wait()
    if guard_sw is None:
        _wait_scatter()
    else:
        pl.when(guard_sw)(_wait_scatter)

    for gp in range(2):
        for c in range(8):
            s16 = c_src[0, pl.ds(c * 16, 16)]
            d16 = c_dst[0, pl.ds(c * 16, 16)]
            c_ixh[gp, pl.ds(c * 16, 16)] = (s16 << 3) + (h0 + gp)
            c_ixo[gp, pl.ds(c * 16, 16)] = d16 + gp * NP

    pltpu.async_copy(inv_hbm.at[c_dst.at[0]], c_inv, c_w2)
    for gp in range(2):
        pltpu.async_copy(hview_hbm.at[c_ixh.at[gp]], c_hr.at[gp], c_w2)

    def _consume():
        pltpu.make_async_copy(inv_hbm.at[n_dst.at[0]], n_inv, n_w2).wait()
        for gp in range(2):
            pltpu.make_async_copy(hview_hbm.at[n_ixh.at[gp]], n_hr.at[gp],
                                  n_w2).wait()
        for v in range(64):
            row = row_half + 2 * v
            xv = plsc.load_gather(n_ex, [row, col8])
            iv = plsc.load_gather(n_inv, [row, col8])
            plsc.store_scatter(n_ex, [row, col8], xv * iv)

        @plsc.parallel_loop(0, 128, 1, unroll=2)
        def _scale(j):
            for gp in range(2):
                w = plsc.load_gather(
                    n_ex, [jnp.full((16,), j, jnp.int32),
                           jnp.full((16,), h0 + gp, jnp.int32)])
                for c in range(4):
                    sl = n_hr[gp, j, pl.ds(c * 16, 16)]
                    n_hr[gp, j, pl.ds(c * 16, 16)] = sl * w

        for gp in range(2):
            pltpu.async_copy(n_hr.at[gp], acc.at[n_ixo.at[gp]], n_s, add=True)

    if guard_cons is None:
        _consume()
    else:
        pl.when(guard_cons)(_consume)

    bn = jnp.minimum(b + 1, NBLK - 1)
    pltpu.async_copy(srcv_hbm.at[wid, bn], n_src.at[0], n_w1)
    pltpu.async_copy(dstv_hbm.at[wid, bn], n_dst.at[0], n_w1)
    pltpu.async_copy(ex_hbm.at[wid, bn], n_ex, n_w1)


def _agg_body(hview_hbm, ex_hbm, inv_hbm, srcv_hbm, dstv_hbm, zbig_hbm,
              part_hbm, acc,
              srcA, dstA, ixhA, ixoA, exA, invA, hrA, w1A, w2A, sA,
              srcB, dstB, ixhB, ixoB, exB, invB, hrB, w1B, w2B, sB):
    cid = lax.axis_index("c")
    sid = lax.axis_index("s")
    wid = sid * 2 + cid

    iota = lax.iota(jnp.int32, 16)
    row_half = iota >> 3
    col8 = iota & 7

    A = (srcA, dstA, ixhA, ixoA, exA, invA, hrA, w1A, w2A, sA)
    B = (srcB, dstB, ixhB, ixoB, exB, invB, hrB, w1B, w2B, sB)

    for g in range(len(H0S)):
        h0 = H0S[g]

        for gp in range(2):
            pltpu.sync_copy(zbig_hbm,
                            acc.at[pl.ds(gp * NP + sid * NPT, NPT)])
        plsc.subcore_barrier()

        pltpu.async_copy(srcv_hbm.at[wid, 0], srcA.at[0], w1A)
        pltpu.async_copy(dstv_hbm.at[wid, 0], dstA.at[0], w1A)
        pltpu.async_copy(ex_hbm.at[wid, 0], exA, w1A)

        def pair(k, _, h0=h0):
            _half_agg(hview_hbm, ex_hbm, inv_hbm, srcv_hbm, dstv_hbm,
                      part_hbm, acc, A, B, wid, 2 * k, h0,
                      k > 0, k > 0, row_half, col8)
            _half_agg(hview_hbm, ex_hbm, inv_hbm, srcv_hbm, dstv_hbm,
                      part_hbm, acc, B, A, wid, 2 * k + 1, h0,
                      k > 0, None, row_half, col8)
            return _

        lax.fori_loop(0, NBLK // 2, pair, None)

        pltpu.make_async_copy(inv_hbm.at[dstB.at[0]], invB, w2B).wait()
        for gp in range(2):
            pltpu.make_async_copy(hview_hbm.at[ixhB.at[gp]], hrB.at[gp],
                                  w2B).wait()
        for v in range(64):
            row = row_half + 2 * v
            xv = plsc.load_gather(exB, [row, col8])
            iv = plsc.load_gather(invB, [row, col8])
            plsc.store_scatter(exB, [row, col8], xv * iv)

        @plsc.parallel_loop(0, 128, 1, unroll=2)
        def _scale_tail(j, h0=h0):
            for gp in range(2):
                w = plsc.load_gather(
                    exB, [jnp.full((16,), j, jnp.int32),
                          jnp.full((16,), h0 + gp, jnp.int32)])
                for c in range(4):
                    sl = hrB[gp, j, pl.ds(c * 16, 16)]
                    hrB[gp, j, pl.ds(c * 16, 16)] = sl * w

        for gp in range(2):
            pltpu.async_copy(hrB.at[gp], acc.at[ixoB.at[gp]], sB, add=True)

        pltpu.make_async_copy(srcv_hbm.at[wid, 0], srcA.at[0], w1A).wait()
        pltpu.make_async_copy(dstv_hbm.at[wid, 0], dstA.at[0], w1A).wait()
        pltpu.make_async_copy(ex_hbm.at[wid, 0], exA, w1A).wait()
        for gp in range(2):
            pltpu.make_async_copy(hrA.at[gp], acc.at[ixoA.at[gp]], sA).wait()
        for gp in range(2):
            pltpu.make_async_copy(hrB.at[gp], acc.at[ixoB.at[gp]], sB).wait()

        plsc.subcore_barrier()
        for gp in range(2):
            pltpu.sync_copy(
                acc.at[pl.ds(gp * NP + sid * NPT, NPT)],
                part_hbm.at[cid, h0 + gp, pl.ds(sid * NPT, NPT)])
        plsc.subcore_barrier()


def _agg(hview, ex, inv16, srcv, dstv, zbig):
    buf = lambda: [
        pltpu.VMEM((1, 128), jnp.int32),
        pltpu.VMEM((1, 128), jnp.int32),
        pltpu.VMEM((2, 128), jnp.int32),
        pltpu.VMEM((2, 128), jnp.int32),
        pltpu.VMEM((128, 16), jnp.float32),
        pltpu.VMEM((128, 16), jnp.float32),
        pltpu.VMEM((2, 128, HID), jnp.float32),
        pltpu.SemaphoreType.DMA,
        pltpu.SemaphoreType.DMA,
        pltpu.SemaphoreType.DMA,
    ]
    return pl.kernel(
        _agg_body,
        out_type=jax.ShapeDtypeStruct((2, HEADS, NP, HID), jnp.float32),
        mesh=_MESH,
        compiler_params=pltpu.CompilerParams(needs_layout_passes=False, use_tc_tiling_on_sc=False),
        scratch_types=[pltpu.VMEM_SHARED((2 * NP, HID), jnp.float32)]
        + buf() + buf(),
    )(hview, ex, inv16, srcv, dstv, zbig)


def _make_asd(a_s, a_d):
    eye = jnp.eye(HEADS, dtype=jnp.float32)
    As = (eye[:, None, :] * a_s[:, :, None]).reshape(HEADS * HID, HEADS)
    Ad = (eye[:, None, :] * a_d[:, :, None]).reshape(HEADS * HID, HEADS)
    return jnp.concatenate([As, Ad], axis=1)


def _layer_edges(h, asd, srcv, dstv, z16, zbig):
    ex, parts = _attn(asd, srcv, dstv, z16)
    inv16 = _inv(parts)
    return _agg(h.reshape(NP * HEADS, HID), ex, inv16, srcv, dstv, zbig)


def kernel(x, edge_index, W1, a1_s, a1_d, b1, W2, a2_s, a2_d, b2,
           W3, a3_s, a3_d, b3, Wp, bp):
    padn = N + jnp.arange(EPAD - E, dtype=jnp.int32) % (NP - N)
    src = jnp.concatenate([edge_index[0], padn]).reshape(NW, NBLK, 128)
    dst = jnp.concatenate([edge_index[1], padn]).reshape(NW, NBLK, 128)
    z16 = jnp.zeros((NPT, 16), jnp.float32)
    zbig = jnp.zeros((NPT, HID), jnp.float32)
    xp = jnp.pad(x, ((0, NP - N), (0, 0)))

    h1, al1 = _mm1(xp, W1, _make_asd(a1_s, a1_d))
    p1 = _layer_edges(h1, al1, src, dst, z16, zbig)

    h2, al2 = _mm2(p1, b1.reshape(HEADS, HID), W2.reshape(HEADS, HID, 512),
                   _make_asd(a2_s, a2_d))
    p2 = _layer_edges(h2, al2, src, dst, z16, zbig)

    h3, al3 = _mm2(p2, b2.reshape(HEADS, HID), W3.reshape(HEADS, HID, 512),
                   _make_asd(a3_s, a3_d))
    p3 = _layer_edges(h3, al3, src, dst, z16, zbig)

    out = _proj(p3, b3, Wp, bp)
    return out[:N]


# speedup vs baseline: 39.0928x; 1.0461x over previous
import functools

import jax
import jax.numpy as jnp
from jax import lax
from jax.experimental import pallas as pl
from jax.experimental.pallas import tpu as pltpu
from jax.experimental.pallas import tpu_sc as plsc

N = 10000
NP = 10240
E = 320000
EPAD = 327680
HEADS = 8
HID = 64
BN = 256
NW = 32
NBLK = 80
NPT = NP // 16
H0S = (0, 2, 4, 6)
GHNS = (2, 2, 2, 2)


def _mm1_body(x_ref, w_ref, asd_ref, h_ref, al_ref):
    h = jnp.dot(x_ref[...], w_ref[...], preferred_element_type=jnp.float32)
    h_ref[...] = h
    al_ref[...] = jnp.dot(h, asd_ref[...], preferred_element_type=jnp.float32)


def _mm1(xp, W, Asd):
    K = xp.shape[1]
    return pl.pallas_call(
        _mm1_body,
        grid=(NP // BN,),
        in_specs=[
            pl.BlockSpec((BN, K), lambda i: (i, 0)),
            pl.BlockSpec((K, 512), lambda i: (0, 0)),
            pl.BlockSpec((512, 16), lambda i: (0, 0)),
        ],
        out_specs=[
            pl.BlockSpec((BN, 512), lambda i: (i, 0)),
            pl.BlockSpec((BN, 16), lambda i: (i, 0)),
        ],
        out_shape=[
            jax.ShapeDtypeStruct((NP, 512), jnp.float32),
            jax.ShapeDtypeStruct((NP, 16), jnp.float32),
        ],
    )(xp, W, Asd)


def _mm2_body(part_ref, b_ref, w_ref, asd_ref, h_ref, al_ref):
    acc = jnp.zeros((BN, 512), jnp.float32)
    for hd in range(HEADS):
        p = part_ref[0, hd] + part_ref[1, hd] + b_ref[hd][None, :]
        xh = jnp.where(p > 0, p, jnp.exp(jnp.minimum(p, 0.0)) - 1.0)
        acc = acc + jnp.dot(xh, w_ref[hd], preferred_element_type=jnp.float32)
    h_ref[...] = acc
    al_ref[...] = jnp.dot(acc, asd_ref[...], preferred_element_type=jnp.float32)


def _mm2(part, b_prev, W, Asd):
    return pl.pallas_call(
        _mm2_body,
        grid=(NP // BN,),
        in_specs=[
            pl.BlockSpec((2, HEADS, BN, HID), lambda i: (0, 0, i, 0)),
            pl.BlockSpec((HEADS, HID), lambda i: (0, 0)),
            pl.BlockSpec((HEADS, HID, 512), lambda i: (0, 0, 0)),
            pl.BlockSpec((512, 16), lambda i: (0, 0)),
        ],
        out_specs=[
            pl.BlockSpec((BN, 512), lambda i: (i, 0)),
            pl.BlockSpec((BN, 16), lambda i: (i, 0)),
        ],
        out_shape=[
            jax.ShapeDtypeStruct((NP, 512), jnp.float32),
            jax.ShapeDtypeStruct((NP, 16), jnp.float32),
        ],
    )(part, b_prev, W, Asd)


def _inv_body(p_ref, o_ref):
    o_ref[...] = 1.0 / (p_ref[0] + p_ref[1] + 1e-16)


def _inv(part):
    p = part.reshape(2, NP // 16, 256)
    out = pl.pallas_call(
        _inv_body,
        grid=(5,),
        in_specs=[pl.BlockSpec((2, NP // 80, 256), lambda i: (0, i, 0))],
        out_specs=pl.BlockSpec((NP // 80, 256), lambda i: (i, 0)),
        out_shape=jax.ShapeDtypeStruct((NP // 16, 256), jnp.float32),
    )(p)
    return out.reshape(NP, 16)


def _proj_body(part_ref, b3_ref, w_ref, bp_ref, o_ref):
    x3 = jnp.zeros((BN, HID), jnp.float32)
    for hd in range(HEADS):
        x3 = x3 + part_ref[0, hd] + part_ref[1, hd]
    x3 = x3 * (1.0 / HEADS) + b3_ref[...]
    o_ref[...] = (
        jnp.dot(x3, w_ref[...], preferred_element_type=jnp.float32)
        + bp_ref[...]
    )


def _proj(part, b3, Wp, bp):
    OUT = Wp.shape[1]
    return pl.pallas_call(
        _proj_body,
        grid=(NP // BN,),
        in_specs=[
            pl.BlockSpec((2, HEADS, BN, HID), lambda i: (0, 0, i, 0)),
            pl.BlockSpec((1, HID), lambda i: (0, 0)),
            pl.BlockSpec((HID, OUT), lambda i: (0, 0)),
            pl.BlockSpec((1, OUT), lambda i: (0, 0)),
        ],
        out_specs=pl.BlockSpec((BN, OUT), lambda i: (i, 0)),
        out_shape=jax.ShapeDtypeStruct((NP, OUT), jnp.float32),
    )(part, b3.reshape(1, HID), Wp, bp.reshape(1, OUT))


_MESH = plsc.VectorSubcoreMesh(
    core_axis_name="c", subcore_axis_name="s", num_cores=2, num_subcores=16)


def _attn_body(asd_hbm, srcv_hbm, dstv_hbm, z16_hbm,
               ex_hbm, part_hbm,
               denom_acc, srcb, dstb, srows, drows, exb, sem):
    cid = lax.axis_index("c")
    sid = lax.axis_index("s")
    wid = sid * 2 + cid

    iota = lax.iota(jnp.int32, 16)
    row_half = iota >> 3
    col8 = iota & 7
    zeros16 = jnp.zeros((16,), jnp.float32)

    pltpu.sync_copy(srcv_hbm.at[wid], srcb)
    pltpu.sync_copy(dstv_hbm.at[wid], dstb)

    for v in range(64):
        plsc.store_scatter(exb, [row_half + 2 * v, col8 + 8], zeros16)

    pltpu.sync_copy(z16_hbm, denom_acc.at[pl.ds(sid * NPT, NPT)])
    plsc.subcore_barrier()

    def block(b, _):
        pltpu.async_copy(asd_hbm.at[srcb.at[b]], srows, sem).wait()
        pltpu.async_copy(asd_hbm.at[dstb.at[b]], drows, sem).wait()
        for v in range(64):
            row = row_half + 2 * v
            es = plsc.load_gather(srows, [row, col8])
            ed = plsc.load_gather(drows, [row, col8 + 8])
            e = es + ed
            e = jnp.where(e > 0, e, 0.2 * e)
            plsc.store_scatter(exb, [row, col8], jnp.exp(e))
        pltpu.sync_copy(exb, ex_hbm.at[wid, b])
        pltpu.sync_copy(exb, denom_acc.at[dstb.at[b]], add=True)
        return _

    lax.fori_loop(0, NBLK, block, None)

    plsc.subcore_barrier()
    pltpu.sync_copy(denom_acc.at[pl.ds(sid * NPT, NPT)],
                    part_hbm.at[cid, pl.ds(sid * NPT, NPT)])


def _attn(asd, srcv, dstv, z16):
    return pl.kernel(
        _attn_body,
        out_type=[
            jax.ShapeDtypeStruct((NW, NBLK, 128, 16), jnp.float32),
            jax.ShapeDtypeStruct((2, NP, 16), jnp.float32),
        ],
        mesh=_MESH,
        compiler_params=pltpu.CompilerParams(needs_layout_passes=False, use_tc_tiling_on_sc=False),
        scratch_types=[
            pltpu.VMEM_SHARED((NP, 16), jnp.float32),
            pltpu.VMEM((NBLK, 128), jnp.int32),
            pltpu.VMEM((NBLK, 128), jnp.int32),
            pltpu.VMEM((128, 16), jnp.float32),
            pltpu.VMEM((128, 16), jnp.float32),
            pltpu.VMEM((128, 16), jnp.float32),
            pltpu.SemaphoreType.DMA,
        ],
    )(asd, srcv, dstv, z16)


def _half_agg(hview_hbm, ex_hbm, inv2_hbm, srcv_hbm, dstv_hbm, acc,
              CUR, NXT, wid, cid, b, h0, guard_sw, guard_cons,
              row_half, col8, row8, col2):
    (c_src, c_dst, c_didx, c_ixh, c_ixo, c_ex, c_inv, c_hr,
     c_w1, c_w2, c_s) = CUR
    (n_src, n_dst, n_didx, n_ixh, n_ixo, n_ex, n_inv, n_hr,
     n_w1, n_w2, n_s) = NXT

    pltpu.make_async_copy(srcv_hbm.at[wid, 0], c_src.at[0], c_w1).wait()
    pltpu.make_async_copy(dstv_hbm.at[wid, 0], c_dst.at[0], c_w1).wait()
    pltpu.make_async_copy(ex_hbm.at[wid, 0], c_ex, c_w1).wait()

    def _wait_scatter():
        for gp in range(2):
            pltpu.make_async_copy(c_hr.at[gp], acc.at[c_ixo.at[gp]], c_s).wait()
    if guard_sw is None:
        _wait_scatter()
    else:
        pl.when(guard_sw)(_wait_scatter)

    for c in range(8):
        s16 = c_src[0, pl.ds(c * 16, 16)]
        d16 = c_dst[0, pl.ds(c * 16, 16)]
        c_didx[0, pl.ds(c * 16, 16)] = d16 + cid * NP
        for gp in range(2):
            c_ixh[gp, pl.ds(c * 16, 16)] = (s16 << 3) + (h0 + gp)
            c_ixo[gp, pl.ds(c * 16, 16)] = d16 + gp * NP

    pltpu.async_copy(inv2_hbm.at[c_didx.at[0]], c_inv, c_w2)
    for gp in range(2):
        pltpu.async_copy(hview_hbm.at[c_ixh.at[gp]], c_hr.at[gp], c_w2)

    def _consume():
        pltpu.make_async_copy(inv2_hbm.at[n_didx.at[0]], n_inv, n_w2).wait()
        for gp in range(2):
            pltpu.make_async_copy(hview_hbm.at[n_ixh.at[gp]], n_hr.at[gp],
                                  n_w2).wait()
        for v in range(16):
            row = row8 + 8 * v
            colw = col2 + h0
            xv = plsc.load_gather(n_ex, [row, colw])
            iv = plsc.load_gather(n_inv, [row, colw])
            plsc.store_scatter(n_ex, [row, colw], xv * iv)

        @plsc.parallel_loop(0, 128, 1, unroll=2)
        def _scale(j):
            for gp in range(2):
                w = plsc.load_gather(
                    n_ex, [jnp.full((16,), j, jnp.int32),
                           jnp.full((16,), h0 + gp, jnp.int32)])
                for c in range(4):
                    sl = n_hr[gp, j, pl.ds(c * 16, 16)]
                    n_hr[gp, j, pl.ds(c * 16, 16)] = sl * w

        for gp in range(2):
            pltpu.async_copy(n_hr.at[gp], acc.at[n_ixo.at[gp]], n_s, add=True)

    if guard_cons is None:
        _consume()
    else:
        pl.when(guard_cons)(_consume)

    bn = jnp.minimum(b + 1, NBLK - 1)
    pltpu.async_copy(srcv_hbm.at[wid, bn], n_src.at[0], n_w1)
    pltpu.async_copy(dstv_hbm.at[wid, bn], n_dst.at[0], n_w1)
    pltpu.async_copy(ex_hbm.at[wid, bn], n_ex, n_w1)


def _agg_body(hview_hbm, ex_hbm, pin_hbm, srcv_hbm, dstv_hbm, zbig_hbm,
              part_hbm, inv2_hbm, acc,
              srcA, dstA, didxA, ixhA, ixoA, exA, invA, hrA, w1A, w2A, sA,
              srcB, dstB, didxB, ixhB, ixoB, exB, invB, hrB, w1B, w2B, sB):
    cid = lax.axis_index("c")
    sid = lax.axis_index("s")
    wid = sid * 2 + cid

    iota = lax.iota(jnp.int32, 16)
    row_half = iota >> 3
    col8 = iota & 7
    row8 = iota >> 1
    col2 = iota & 1

    A = (srcA, dstA, didxA, ixhA, ixoA, exA, invA, hrA, w1A, w2A, sA)
    B = (srcB, dstB, didxB, ixhB, ixoB, exB, invB, hrB, w1B, w2B, sB)

    base = cid * NP + sid * NPT
    for ch in range(5):
        pltpu.sync_copy(pin_hbm.at[0, pl.ds(sid * NPT + ch * 128, 128)], exA)
        pltpu.sync_copy(pin_hbm.at[1, pl.ds(sid * NPT + ch * 128, 128)], invA)

        def invrow(v, _):
            t = exA[v, :] + invA[v, :]
            exA[v, :] = 1.0 / (t + 1e-16)
            return _

        lax.fori_loop(0, 128, invrow, None)
        pltpu.sync_copy(exA, inv2_hbm.at[pl.ds(base + ch * 128, 128)])
    plsc.subcore_barrier()

    for g in range(len(H0S)):
        h0 = H0S[g]

        for gp in range(2):
            pltpu.sync_copy(zbig_hbm,
                            acc.at[pl.ds(gp * NP + sid * NPT, NPT)])
        plsc.subcore_barrier()

        pltpu.async_copy(srcv_hbm.at[wid, 0], srcA.at[0], w1A)
        pltpu.async_copy(dstv_hbm.at[wid, 0], dstA.at[0], w1A)
        pltpu.async_copy(ex_hbm.at[wid, 0], exA, w1A)

        def pair(k, _, h0=h0):
            _half_agg(hview_hbm, ex_hbm, inv2_hbm, srcv_hbm, dstv_hbm,
                      acc, A, B, wid, cid, 2 * k, h0,
                      k > 0, k > 0, row_half, col8, row8, col2)
            _half_agg(hview_hbm, ex_hbm, inv2_hbm, srcv_hbm, dstv_hbm,
                      acc, B, A, wid, cid, 2 * k + 1, h0,
                      k > 0, None, row_half, col8, row8, col2)
            return _

        lax.fori_loop(0, NBLK // 2, pair, None)

        pltpu.make_async_copy(inv2_hbm.at[didxB.at[0]], invB, w2B).wait()
        for gp in range(2):
            pltpu.make_async_copy(hview_hbm.at[ixhB.at[gp]], hrB.at[gp],
                                  w2B).wait()
        for v in range(16):
            row = row8 + 8 * v
            colw = col2 + h0
            xv = plsc.load_gather(exB, [row, colw])
            iv = plsc.load_gather(invB, [row, colw])
            plsc.store_scatter(exB, [row, colw], xv * iv)

        @plsc.parallel_loop(0, 128, 1, unroll=2)
        def _scale_tail(j, h0=h0):
            for gp in range(2):
                w = plsc.load_gather(
                    exB, [jnp.full((16,), j, jnp.int32),
                          jnp.full((16,), h0 + gp, jnp.int32)])
                for c in range(4):
                    sl = hrB[gp, j, pl.ds(c * 16, 16)]
                    hrB[gp, j, pl.ds(c * 16, 16)] = sl * w

        for gp in range(2):
            pltpu.async_copy(hrB.at[gp], acc.at[ixoB.at[gp]], sB, add=True)

        pltpu.make_async_copy(srcv_hbm.at[wid, 0], srcA.at[0], w1A).wait()
        pltpu.make_async_copy(dstv_hbm.at[wid, 0], dstA.at[0], w1A).wait()
        pltpu.make_async_copy(ex_hbm.at[wid, 0], exA, w1A).wait()
        for gp in range(2):
            pltpu.make_async_copy(hrA.at[gp], acc.at[ixoA.at[gp]], sA).wait()
        for gp in range(2):
            pltpu.make_async_copy(hrB.at[gp], acc.at[ixoB.at[gp]], sB).wait()

        plsc.subcore_barrier()
        for gp in range(2):
            pltpu.sync_copy(
                acc.at[pl.ds(gp * NP + sid * NPT, NPT)],
                part_hbm.at[cid, h0 + gp, pl.ds(sid * NPT, NPT)])
        plsc.subcore_barrier()


def _agg(hview, ex, parts, srcv, dstv, zbig):
    buf = lambda: [
        pltpu.VMEM((1, 128), jnp.int32),
        pltpu.VMEM((1, 128), jnp.int32),
        pltpu.VMEM((1, 128), jnp.int32),
        pltpu.VMEM((2, 128), jnp.int32),
        pltpu.VMEM((2, 128), jnp.int32),
        pltpu.VMEM((128, 16), jnp.float32),
        pltpu.VMEM((128, 16), jnp.float32),
        pltpu.VMEM((2, 128, HID), jnp.float32),
        pltpu.SemaphoreType.DMA,
        pltpu.SemaphoreType.DMA,
        pltpu.SemaphoreType.DMA,
    ]
    out = pl.kernel(
        _agg_body,
        out_type=[
            jax.ShapeDtypeStruct((2, HEADS, NP, HID), jnp.float32),
            jax.ShapeDtypeStruct((2 * NP, 16), jnp.float32),
        ],
        mesh=_MESH,
        compiler_params=pltpu.CompilerParams(needs_layout_passes=False, use_tc_tiling_on_sc=False),
        scratch_types=[pltpu.VMEM_SHARED((2 * NP, HID), jnp.float32)]
        + buf() + buf(),
    )(hview, ex, parts, srcv, dstv, zbig)
    return out[0]


def _make_asd(a_s, a_d):
    eye = jnp.eye(HEADS, dtype=jnp.float32)
    As = (eye[:, None, :] * a_s[:, :, None]).reshape(HEADS * HID, HEADS)
    Ad = (eye[:, None, :] * a_d[:, :, None]).reshape(HEADS * HID, HEADS)
    return jnp.concatenate([As, Ad], axis=1)


def _layer_edges(h, asd, srcv, dstv, z16, zbig):
    ex, parts = _attn(asd, srcv, dstv, z16)
    return _agg(h.reshape(NP * HEADS, HID), ex, parts, srcv, dstv, zbig)


def kernel(x, edge_index, W1, a1_s, a1_d, b1, W2, a2_s, a2_d, b2,
           W3, a3_s, a3_d, b3, Wp, bp):
    padn = N + jnp.arange(EPAD - E, dtype=jnp.int32) % (NP - N)
    src = jnp.concatenate([edge_index[0], padn]).reshape(NW, NBLK, 128)
    dst = jnp.concatenate([edge_index[1], padn]).reshape(NW, NBLK, 128)
    z16 = jnp.zeros((NPT, 16), jnp.float32)
    zbig = jnp.zeros((NPT, HID), jnp.float32)
    xp = jnp.pad(x, ((0, NP - N), (0, 0)))

    h1, al1 = _mm1(xp, W1, _make_asd(a1_s, a1_d))
    p1 = _layer_edges(h1, al1, src, dst, z16, zbig)

    h2, al2 = _mm2(p1, b1.reshape(HEADS, HID), W2.reshape(HEADS, HID, 512),
                   _make_asd(a2_s, a2_d))
    p2 = _layer_edges(h2, al2, src, dst, z16, zbig)

    h3, al3 = _mm2(p2, b2.reshape(HEADS, HID), W3.reshape(HEADS, HID, 512),
                   _make_asd(a3_s, a3_d))
    p3 = _layer_edges(h3, al3, src, dst, z16, zbig)

    out = _proj(p3, b3, Wp, bp)
    return out[:N]


# speedup vs baseline: 43.6118x; 1.1156x over previous
import functools

import jax
import jax.numpy as jnp
from jax import lax
from jax.experimental import pallas as pl
from jax.experimental.pallas import tpu as pltpu
from jax.experimental.pallas import tpu_sc as plsc

N = 10000
NP = 10240
E = 320000
EPAD = 327680
HEADS = 8
HID = 64
BN = 256
NW = 32
NBLK = 80
NPT = NP // 16
H0S = (0, 2, 4, 6)
GHNS = (2, 2, 2, 2)


def _mm1_body(x_ref, w_ref, asd_ref, h_ref, al_ref):
    h = jnp.dot(x_ref[...], w_ref[...], preferred_element_type=jnp.float32)
    h_ref[...] = h
    al_ref[...] = jnp.dot(h, asd_ref[...], preferred_element_type=jnp.float32)


def _mm1(xp, W, Asd):
    K = xp.shape[1]
    return pl.pallas_call(
        _mm1_body,
        grid=(NP // BN,),
        in_specs=[
            pl.BlockSpec((BN, K), lambda i: (i, 0)),
            pl.BlockSpec((K, 512), lambda i: (0, 0)),
            pl.BlockSpec((512, 16), lambda i: (0, 0)),
        ],
        out_specs=[
            pl.BlockSpec((BN, 512), lambda i: (i, 0)),
            pl.BlockSpec((BN, 16), lambda i: (i, 0)),
        ],
        out_shape=[
            jax.ShapeDtypeStruct((NP, 512), jnp.float32),
            jax.ShapeDtypeStruct((NP, 16), jnp.float32),
        ],
    )(xp, W, Asd)


def _mm2_body(part_ref, b_ref, w_ref, asd_ref, h_ref, al_ref):
    acc = jnp.zeros((BN, 512), jnp.float32)
    for hd in range(HEADS):
        p = part_ref[0, hd] + part_ref[1, hd] + b_ref[hd][None, :]
        xh = jnp.where(p > 0, p, jnp.exp(jnp.minimum(p, 0.0)) - 1.0)
        acc = acc + jnp.dot(xh, w_ref[hd], preferred_element_type=jnp.float32)
    h_ref[...] = acc
    al_ref[...] = jnp.dot(acc, asd_ref[...], preferred_element_type=jnp.float32)


def _mm2(part, b_prev, W, Asd):
    return pl.pallas_call(
        _mm2_body,
        grid=(NP // BN,),
        in_specs=[
            pl.BlockSpec((2, HEADS, BN, HID), lambda i: (0, 0, i, 0)),
            pl.BlockSpec((HEADS, HID), lambda i: (0, 0)),
            pl.BlockSpec((HEADS, HID, 512), lambda i: (0, 0, 0)),
            pl.BlockSpec((512, 16), lambda i: (0, 0)),
        ],
        out_specs=[
            pl.BlockSpec((BN, 512), lambda i: (i, 0)),
            pl.BlockSpec((BN, 16), lambda i: (i, 0)),
        ],
        out_shape=[
            jax.ShapeDtypeStruct((NP, 512), jnp.float32),
            jax.ShapeDtypeStruct((NP, 16), jnp.float32),
        ],
    )(part, b_prev, W, Asd)


def _inv_body(p_ref, o_ref):
    o_ref[...] = 1.0 / (p_ref[0] + p_ref[1] + 1e-16)


def _inv(part):
    p = part.reshape(2, NP // 16, 256)
    out = pl.pallas_call(
        _inv_body,
        grid=(5,),
        in_specs=[pl.BlockSpec((2, NP // 80, 256), lambda i: (0, i, 0))],
        out_specs=pl.BlockSpec((NP // 80, 256), lambda i: (i, 0)),
        out_shape=jax.ShapeDtypeStruct((NP // 16, 256), jnp.float32),
    )(p)
    return out.reshape(NP, 16)


def _proj_body(part_ref, b3_ref, w_ref, bp_ref, o_ref):
    x3 = jnp.zeros((BN, HID), jnp.float32)
    for hd in range(HEADS):
        x3 = x3 + part_ref[0, hd] + part_ref[1, hd]
    x3 = x3 * (1.0 / HEADS) + b3_ref[...]
    o_ref[...] = (
        jnp.dot(x3, w_ref[...], preferred_element_type=jnp.float32)
        + bp_ref[...]
    )


def _proj(part, b3, Wp, bp):
    OUT = Wp.shape[1]
    return pl.pallas_call(
        _proj_body,
        grid=(NP // BN,),
        in_specs=[
            pl.BlockSpec((2, HEADS, BN, HID), lambda i: (0, 0, i, 0)),
            pl.BlockSpec((1, HID), lambda i: (0, 0)),
            pl.BlockSpec((HID, OUT), lambda i: (0, 0)),
            pl.BlockSpec((1, OUT), lambda i: (0, 0)),
        ],
        out_specs=pl.BlockSpec((BN, OUT), lambda i: (i, 0)),
        out_shape=jax.ShapeDtypeStruct((NP, OUT), jnp.float32),
    )(part, b3.reshape(1, HID), Wp, bp.reshape(1, OUT))


_MESH = plsc.VectorSubcoreMesh(
    core_axis_name="c", subcore_axis_name="s", num_cores=2, num_subcores=16)


def _attn_step(asd_hbm, ex_hbm, dacc, srcb, dstb,
               sr_c, dr_c, ex_c, w2_c, swr_c, ssc_c, sr_n, dr_n, w2_n,
               wid, b, g_sw, row_half, col8):
    pltpu.make_async_copy(asd_hbm.at[srcb.at[b]], sr_c, w2_c).wait()
    pltpu.make_async_copy(asd_hbm.at[dstb.at[b]], dr_c, w2_c).wait()
    bn = jnp.minimum(b + 1, NBLK - 1)
    pltpu.async_copy(asd_hbm.at[srcb.at[bn]], sr_n, w2_n)
    pltpu.async_copy(asd_hbm.at[dstb.at[bn]], dr_n, w2_n)

    def _wait_s():
        pltpu.make_async_copy(ex_c, ex_hbm.at[wid, 0], swr_c).wait()
        pltpu.make_async_copy(ex_c, dacc.at[dstb.at[b]], ssc_c).wait()
    if g_sw is None:
        _wait_s()
    else:
        pl.when(g_sw)(_wait_s)

    for v in range(64):
        row = row_half + 2 * v
        es = plsc.load_gather(sr_c, [row, col8])
        ed = plsc.load_gather(dr_c, [row, col8 + 8])
        e = es + ed
        e = jnp.where(e > 0, e, 0.2 * e)
        plsc.store_scatter(ex_c, [row, col8], jnp.exp(e))
    pltpu.async_copy(ex_c, ex_hbm.at[wid, b], swr_c)
    pltpu.async_copy(ex_c, dacc.at[dstb.at[b]], ssc_c, add=True)


def _attn_body(asd_hbm, srcv_hbm, dstv_hbm, z16_hbm,
               ex_hbm, part_hbm,
               dacc, srcb, dstb, srA, drA, exA, w2A, swrA, sscA,
               srB, drB, exB, w2B, swrB, sscB):
    cid = lax.axis_index("c")
    sid = lax.axis_index("s")
    wid = sid * 2 + cid

    iota = lax.iota(jnp.int32, 16)
    row_half = iota >> 3
    col8 = iota & 7
    zeros16 = jnp.zeros((16,), jnp.float32)

    pltpu.sync_copy(srcv_hbm.at[wid], srcb)
    pltpu.sync_copy(dstv_hbm.at[wid], dstb)

    for v in range(64):
        plsc.store_scatter(exA, [row_half + 2 * v, col8 + 8], zeros16)
        plsc.store_scatter(exB, [row_half + 2 * v, col8 + 8], zeros16)

    pltpu.sync_copy(z16_hbm, dacc.at[pl.ds(sid * NPT, NPT)])
    plsc.subcore_barrier()

    pltpu.async_copy(asd_hbm.at[srcb.at[0]], srA, w2A)
    pltpu.async_copy(asd_hbm.at[dstb.at[0]], drA, w2A)

    def pair(k, _):
        _attn_step(asd_hbm, ex_hbm, dacc, srcb, dstb,
                   srA, drA, exA, w2A, swrA, sscA, srB, drB, w2B,
                   wid, 2 * k, k > 0, row_half, col8)
        _attn_step(asd_hbm, ex_hbm, dacc, srcb, dstb,
                   srB, drB, exB, w2B, swrB, sscB, srA, drA, w2A,
                   wid, 2 * k + 1, k > 0, row_half, col8)
        return _

    lax.fori_loop(0, NBLK // 2, pair, None)

    pltpu.make_async_copy(asd_hbm.at[srcb.at[0]], srA, w2A).wait()
    pltpu.make_async_copy(asd_hbm.at[dstb.at[0]], drA, w2A).wait()
    pltpu.make_async_copy(exA, ex_hbm.at[wid, 0], swrA).wait()
    pltpu.make_async_copy(exA, dacc.at[dstb.at[0]], sscA).wait()
    pltpu.make_async_copy(exB, ex_hbm.at[wid, 0], swrB).wait()
    pltpu.make_async_copy(exB, dacc.at[dstb.at[0]], sscB).wait()

    plsc.subcore_barrier()
    pltpu.sync_copy(dacc.at[pl.ds(sid * NPT, NPT)],
                    part_hbm.at[cid, pl.ds(sid * NPT, NPT)])


def _attn(asd, srcv, dstv, z16):
    return pl.kernel(
        _attn_body,
        out_type=[
            jax.ShapeDtypeStruct((NW, NBLK, 128, 16), jnp.float32),
            jax.ShapeDtypeStruct((2, NP, 16), jnp.float32),
        ],
        mesh=_MESH,
        compiler_params=pltpu.CompilerParams(needs_layout_passes=False, use_tc_tiling_on_sc=False),
        scratch_types=[
            pltpu.VMEM_SHARED((NP, 16), jnp.float32),
            pltpu.VMEM((NBLK, 128), jnp.int32),
            pltpu.VMEM((NBLK, 128), jnp.int32),
            pltpu.VMEM((128, 16), jnp.float32),
            pltpu.VMEM((128, 16), jnp.float32),
            pltpu.VMEM((128, 16), jnp.float32),
            pltpu.SemaphoreType.DMA,
            pltpu.SemaphoreType.DMA,
            pltpu.SemaphoreType.DMA,
            pltpu.VMEM((128, 16), jnp.float32),
            pltpu.VMEM((128, 16), jnp.float32),
            pltpu.VMEM((128, 16), jnp.float32),
            pltpu.SemaphoreType.DMA,
            pltpu.SemaphoreType.DMA,
            pltpu.SemaphoreType.DMA,
        ],
    )(asd, srcv, dstv, z16)


def _half_agg(hview_hbm, ex_hbm, inv2_hbm, srcv_hbm, dstv_hbm, acc,
              CUR, NXT, wid, cid, b, h0, guard_sw, guard_cons,
              row_half, col8, row8, col2):
    (c_src, c_dst, c_didx, c_ixh, c_ixo, c_ex, c_inv, c_hr,
     c_w1, c_w2, c_s) = CUR
    (n_src, n_dst, n_didx, n_ixh, n_ixo, n_ex, n_inv, n_hr,
     n_w1, n_w2, n_s) = NXT

    pltpu.make_async_copy(srcv_hbm.at[wid, 0], c_src.at[0], c_w1).wait()
    pltpu.make_async_copy(dstv_hbm.at[wid, 0], c_dst.at[0], c_w1).wait()
    pltpu.make_async_copy(ex_hbm.at[wid, 0], c_ex, c_w1).wait()

    def _wait_scatter():
        for gp in range(2):
            pltpu.make_async_copy(c_hr.at[gp], acc.at[c_ixo.at[gp]], c_s).wait()
    if guard_sw is None:
        _wait_scatter()
    else:
        pl.when(guard_sw)(_wait_scatter)

    for c in range(8):
        s16 = c_src[0, pl.ds(c * 16, 16)]
        d16 = c_dst[0, pl.ds(c * 16, 16)]
        c_didx[0, pl.ds(c * 16, 16)] = d16 + cid * NP
        for gp in range(2):
            c_ixh[gp, pl.ds(c * 16, 16)] = (s16 << 3) + (h0 + gp)
            c_ixo[gp, pl.ds(c * 16, 16)] = d16 + gp * NP

    pltpu.async_copy(inv2_hbm.at[c_didx.at[0]], c_inv, c_w2)
    for gp in range(2):
        pltpu.async_copy(hview_hbm.at[c_ixh.at[gp]], c_hr.at[gp], c_w2)

    def _consume():
        pltpu.make_async_copy(inv2_hbm.at[n_didx.at[0]], n_inv, n_w2).wait()
        for gp in range(2):
            pltpu.make_async_copy(hview_hbm.at[n_ixh.at[gp]], n_hr.at[gp],
                                  n_w2).wait()
        for v in range(16):
            row = row8 + 8 * v
            colw = col2 + h0
            xv = plsc.load_gather(n_ex, [row, colw])
            iv = plsc.load_gather(n_inv, [row, colw])
            plsc.store_scatter(n_ex, [row, colw], xv * iv)

        @plsc.parallel_loop(0, 128, 1, unroll=2)
        def _scale(j):
            for gp in range(2):
                w = plsc.load_gather(
                    n_ex, [jnp.full((16,), j, jnp.int32),
                           jnp.full((16,), h0 + gp, jnp.int32)])
                for c in range(4):
                    sl = n_hr[gp, j, pl.ds(c * 16, 16)]
                    n_hr[gp, j, pl.ds(c * 16, 16)] = sl * w

        for gp in range(2):
            pltpu.async_copy(n_hr.at[gp], acc.at[n_ixo.at[gp]], n_s, add=True)

    if guard_cons is None:
        _consume()
    else:
        pl.when(guard_cons)(_consume)

    bn = jnp.minimum(b + 1, NBLK - 1)
    pltpu.async_copy(srcv_hbm.at[wid, bn], n_src.at[0], n_w1)
    pltpu.async_copy(dstv_hbm.at[wid, bn], n_dst.at[0], n_w1)
    pltpu.async_copy(ex_hbm.at[wid, bn], n_ex, n_w1)


def _agg_body(hview_hbm, ex_hbm, pin_hbm, srcv_hbm, dstv_hbm, zbig_hbm,
              part_hbm, inv2_hbm, acc,
              srcA, dstA, didxA, ixhA, ixoA, exA, invA, hrA, w1A, w2A, sA,
              srcB, dstB, didxB, ixhB, ixoB, exB, invB, hrB, w1B, w2B, sB):
    cid = lax.axis_index("c")
    sid = lax.axis_index("s")
    wid = sid * 2 + cid

    iota = lax.iota(jnp.int32, 16)
    row_half = iota >> 3
    col8 = iota & 7
    row8 = iota >> 1
    col2 = iota & 1

    A = (srcA, dstA, didxA, ixhA, ixoA, exA, invA, hrA, w1A, w2A, sA)
    B = (srcB, dstB, didxB, ixhB, ixoB, exB, invB, hrB, w1B, w2B, sB)

    base = cid * NP + sid * NPT
    for ch in range(5):
        pltpu.sync_copy(pin_hbm.at[0, pl.ds(sid * NPT + ch * 128, 128)], exA)
        pltpu.sync_copy(pin_hbm.at[1, pl.ds(sid * NPT + ch * 128, 128)], invA)

        def invrow(v, _):
            t = exA[v, :] + invA[v, :]
            exA[v, :] = 1.0 / (t + 1e-16)
            return _

        lax.fori_loop(0, 128, invrow, None)
        pltpu.sync_copy(exA, inv2_hbm.at[pl.ds(base + ch * 128, 128)])
    plsc.subcore_barrier()

    for g in range(len(H0S)):
        h0 = H0S[g]

        for gp in range(2):
            pltpu.sync_copy(zbig_hbm,
                            acc.at[pl.ds(gp * NP + sid * NPT, NPT)])
        plsc.subcore_barrier()

        pltpu.async_copy(srcv_hbm.at[wid, 0], srcA.at[0], w1A)
        pltpu.async_copy(dstv_hbm.at[wid, 0], dstA.at[0], w1A)
        pltpu.async_copy(ex_hbm.at[wid, 0], exA, w1A)

        def pair(k, _, h0=h0):
            _half_agg(hview_hbm, ex_hbm, inv2_hbm, srcv_hbm, dstv_hbm,
                      acc, A, B, wid, cid, 2 * k, h0,
                      k > 0, k > 0, row_half, col8, row8, col2)
            _half_agg(hview_hbm, ex_hbm, inv2_hbm, srcv_hbm, dstv_hbm,
                      acc, B, A, wid, cid, 2 * k + 1, h0,
                      k > 0, None, row_half, col8, row8, col2)
            return _

        lax.fori_loop(0, NBLK // 2, pair, None)

        pltpu.make_async_copy(inv2_hbm.at[didxB.at[0]], invB, w2B).wait()
        for gp in range(2):
            pltpu.make_async_copy(hview_hbm.at[ixhB.at[gp]], hrB.at[gp],
                                  w2B).wait()
        for v in range(16):
            row = row8 + 8 * v
            colw = col2 + h0
            xv = plsc.load_gather(exB, [row, colw])
            iv = plsc.load_gather(invB, [row, colw])
            plsc.store_scatter(exB, [row, colw], xv * iv)

        @plsc.parallel_loop(0, 128, 1, unroll=2)
        def _scale_tail(j, h0=h0):
            for gp in range(2):
                w = plsc.load_gather(
                    exB, [jnp.full((16,), j, jnp.int32),
                          jnp.full((16,), h0 + gp, jnp.int32)])
                for c in range(4):
                    sl = hrB[gp, j, pl.ds(c * 16, 16)]
                    hrB[gp, j, pl.ds(c * 16, 16)] = sl * w

        for gp in range(2):
            pltpu.async_copy(hrB.at[gp], acc.at[ixoB.at[gp]], sB, add=True)

        pltpu.make_async_copy(srcv_hbm.at[wid, 0], srcA.at[0], w1A).wait()
        pltpu.make_async_copy(dstv_hbm.at[wid, 0], dstA.at[0], w1A).wait()
        pltpu.make_async_copy(ex_hbm.at[wid, 0], exA, w1A).wait()
        for gp in range(2):
            pltpu.make_async_copy(hrA.at[gp], acc.at[ixoA.at[gp]], sA).wait()
        for gp in range(2):
            pltpu.make_async_copy(hrB.at[gp], acc.at[ixoB.at[gp]], sB).wait()

        plsc.subcore_barrier()
        for gp in range(2):
            pltpu.sync_copy(
                acc.at[pl.ds(gp * NP + sid * NPT, NPT)],
                part_hbm.at[cid, h0 + gp, pl.ds(sid * NPT, NPT)])
        plsc.subcore_barrier()


def _agg(hview, ex, parts, srcv, dstv, zbig):
    buf = lambda: [
        pltpu.VMEM((1, 128), jnp.int32),
        pltpu.VMEM((1, 128), jnp.int32),
        pltpu.VMEM((1, 128), jnp.int32),
        pltpu.VMEM((2, 128), jnp.int32),
        pltpu.VMEM((2, 128), jnp.int32),
        pltpu.VMEM((128, 16), jnp.float32),
        pltpu.VMEM((128, 16), jnp.float32),
        pltpu.VMEM((2, 128, HID), jnp.float32),
        pltpu.SemaphoreType.DMA,
        pltpu.SemaphoreType.DMA,
        pltpu.SemaphoreType.DMA,
    ]
    out = pl.kernel(
        _agg_body,
        out_type=[
            jax.ShapeDtypeStruct((2, HEADS, NP, HID), jnp.float32),
            jax.ShapeDtypeStruct((2 * NP, 16), jnp.float32),
        ],
        mesh=_MESH,
        compiler_params=pltpu.CompilerParams(needs_layout_passes=False, use_tc_tiling_on_sc=False),
        scratch_types=[pltpu.VMEM_SHARED((2 * NP, HID), jnp.float32)]
        + buf() + buf(),
    )(hview, ex, parts, srcv, dstv, zbig)
    return out[0]


def _make_asd(a_s, a_d):
    eye = jnp.eye(HEADS, dtype=jnp.float32)
    As = (eye[:, None, :] * a_s[:, :, None]).reshape(HEADS * HID, HEADS)
    Ad = (eye[:, None, :] * a_d[:, :, None]).reshape(HEADS * HID, HEADS)
    return jnp.concatenate([As, Ad], axis=1)


def _layer_edges(h, asd, srcv, dstv, z16, zbig):
    ex, parts = _attn(asd, srcv, dstv, z16)
    return _agg(h.reshape(NP * HEADS, HID), ex, parts, srcv, dstv, zbig)


def kernel(x, edge_index, W1, a1_s, a1_d, b1, W2, a2_s, a2_d, b2,
           W3, a3_s, a3_d, b3, Wp, bp):
    padn = N + jnp.arange(EPAD - E, dtype=jnp.int32) % (NP - N)
    src = jnp.concatenate([edge_index[0], padn]).reshape(NW, NBLK, 128)
    dst = jnp.concatenate([edge_index[1], padn]).reshape(NW, NBLK, 128)
    z16 = jnp.zeros((NPT, 16), jnp.float32)
    zbig = jnp.zeros((NPT, HID), jnp.float32)
    xp = jnp.pad(x, ((0, NP - N), (0, 0)))

    h1, al1 = _mm1(xp, W1, _make_asd(a1_s, a1_d))
    p1 = _layer_edges(h1, al1, src, dst, z16, zbig)

    h2, al2 = _mm2(p1, b1.reshape(HEADS, HID), W2.reshape(HEADS, HID, 512),
                   _make_asd(a2_s, a2_d))
    p2 = _layer_edges(h2, al2, src, dst, z16, zbig)

    h3, al3 = _mm2(p2, b2.reshape(HEADS, HID), W3.reshape(HEADS, HID, 512),
                   _make_asd(a3_s, a3_d))
    p3 = _layer_edges(h3, al3, src, dst, z16, zbig)

    out = _proj(p3, b3, Wp, bp)
    return out[:N]
